# baseline (speedup 1.0000x reference)
import dataclasses
import functools

import jax
import jax.numpy as jnp
from jax import lax
from jax.experimental import pallas as pl
from jax.experimental.pallas import tpu as pltpu
from jax.experimental.pallas import tpu_sc as plsc

B = 16384
F = 26
V = 100000
D = 16
ND = 13
H = 400
FD = F * D
TPF = D + 1
NT = F * TPF

_NC = 2
_NS = 16
_NW = _NC * _NS
_ICH = 2048
_NCH = B // _ICH


def _sc_compiler_params():
    cp = pltpu.CompilerParams(use_tc_tiling_on_sc=True)
    if "needs_layout_passes" in pltpu.CompilerParams.__dataclass_fields__:
        cp = dataclasses.replace(cp, needs_layout_passes=False)
    return cp


def _sc_gather(tab2d, lin2d, idxT):
    mesh = plsc.VectorSubcoreMesh(core_axis_name="c", subcore_axis_name="s")

    @functools.partial(
        pl.kernel,
        mesh=mesh,
        out_type=(
            jax.ShapeDtypeStruct((FD, B), jnp.float32),
            jax.ShapeDtypeStruct((F, B), jnp.float32),
        ),
        scratch_types=[
            pltpu.VMEM((V,), jnp.float32),
            pltpu.VMEM((B,), jnp.int32),
            pltpu.VMEM((_ICH,), jnp.float32),
            pltpu.VMEM((_ICH,), jnp.float32),
            pltpu.SemaphoreType.DMA,
            pltpu.SemaphoreType.DMA,
            pltpu.SemaphoreType.DMA,
        ],
        compiler_params=_sc_compiler_params(),
    )
    def gather_kernel(tab_hbm, lin_hbm, idx_hbm, embT_out, linT_out,
                      plane_v, idx_v, out_a, out_b, sem_a, sem_b, sem_p):
        wid = lax.axis_index("s") * _NC + lax.axis_index("c")
        lo = wid * NT // _NW
        hi = (wid + 1) * NT // _NW

        def gather_chunk(base, out_v):
            @plsc.parallel_loop(0, _ICH, 16, unroll=8)
            def _(j):
                sl = pl.ds(j, 16)
                out_v[sl] = plsc.load_gather(
                    plane_v, [idx_v[pl.ds(base + j, 16)]])

        def do_plane(out_hbm, out_row, first):
            @pl.loop(0, _NCH, step=2)
            def _(c):
                not_first = jnp.logical_or(c > 0, jnp.logical_not(first))

                @pl.when(not_first)
                def _():
                    pltpu.make_async_copy(
                        out_a, out_hbm.at[out_row, pl.ds(0, _ICH)],
                        sem_a).wait()

                gather_chunk(c * _ICH, out_a)
                pltpu.async_copy(
                    out_a, out_hbm.at[out_row, pl.ds(c * _ICH, _ICH)], sem_a)

                @pl.when(not_first)
                def _():
                    pltpu.make_async_copy(
                        out_b, out_hbm.at[out_row, pl.ds(0, _ICH)],
                        sem_b).wait()

                gather_chunk((c + 1) * _ICH, out_b)
                pltpu.async_copy(
                    out_b, out_hbm.at[out_row, pl.ds((c + 1) * _ICH, _ICH)],
                    sem_b)

        @pl.loop(lo, hi)
        def _(t):
            f = t // TPF
            k = t - f * TPF
            first = t == lo

            @pl.when(jnp.logical_or(first, k == 0))
            def _():
                pltpu.sync_copy(idx_hbm.at[f], idx_v)

            @pl.when(k < D)
            def _():
                pltpu.sync_copy(tab_hbm.at[f * D + k], plane_v)
                do_plane(embT_out, f * D + k, first)

            @pl.when(k == D)
            def _():
                pltpu.sync_copy(lin_hbm.at[f], plane_v)
                do_plane(linT_out, f, first)

        pltpu.make_async_copy(
            out_a, embT_out.at[0, pl.ds(0, _ICH)], sem_a).wait()
        pltpu.make_async_copy(
            out_b, embT_out.at[0, pl.ds(0, _ICH)], sem_b).wait()

    return gather_kernel(tab2d, lin2d, idxT)


_BS = 512


def _tc_body(embT_ref, denseT_ref, linT_ref, w1e_ref, w1d_ref, b1_ref,
             w2_ref, b2_ref, w3t_ref, dw_ref, c0_ref, out_ref):
    eT = embT_ref[...]
    dT = denseT_ref[...]
    dn0 = (((0,), (0,)), ((), ()))
    bf = jnp.bfloat16
    eTb = eT.astype(bf)
    x1 = lax.dot_general(eTb, w1e_ref[...].astype(bf), dn0,
                         preferred_element_type=jnp.float32)
    x1 = x1 + lax.dot_general(dT, w1d_ref[...], dn0,
                              preferred_element_type=jnp.float32)
    h = jnp.tanh(x1 + b1_ref[...])
    h = jnp.tanh(lax.dot_general(
        h.astype(bf), w2_ref[...].astype(bf), (((1,), (0,)), ((), ())),
        preferred_element_type=jnp.float32) + b2_ref[...])
    deep = jnp.sum(h * w3t_ref[...], axis=1)
    r = lax.broadcasted_iota(jnp.int32, (FD, D), 0)
    c = lax.broadcasted_iota(jnp.int32, (FD, D), 1)
    s_mat = jnp.where(lax.rem(r, D) == c, 1.0, 0.0).astype(bf)
    t = lax.dot_general(eTb, s_mat, dn0,
                        preferred_element_type=jnp.float32)
    inter = 0.5 * (jnp.sum(t * t, axis=1) - jnp.sum(eT * eT, axis=0))
    linear = (jnp.sum(linT_ref[...], axis=0)
              + jnp.sum(dT * dw_ref[...], axis=0) + c0_ref[0, 0])
    out_ref[...] = deep + inter + linear


def _tc_forward(embT, denseT, linT, w1e, w1d, b1r, w2, b2r, w3t, dwc, c0):
    return pl.pallas_call(
        _tc_body,
        grid=(B // _BS,),
        in_specs=[
            pl.BlockSpec((FD, _BS), lambda i: (0, i)),
            pl.BlockSpec((ND, _BS), lambda i: (0, i)),
            pl.BlockSpec((F, _BS), lambda i: (0, i)),
            pl.BlockSpec((FD, H), lambda i: (0, 0)),
            pl.BlockSpec((ND, H), lambda i: (0, 0)),
            pl.BlockSpec((1, H), lambda i: (0, 0)),
            pl.BlockSpec((H, H), lambda i: (0, 0)),
            pl.BlockSpec((1, H), lambda i: (0, 0)),
            pl.BlockSpec((1, H), lambda i: (0, 0)),
            pl.BlockSpec((ND, 1), lambda i: (0, 0)),
            pl.BlockSpec((1, 1), lambda i: (0, 0)),
        ],
        out_specs=pl.BlockSpec((_BS,), lambda i: (i,)),
        out_shape=jax.ShapeDtypeStruct((B,), jnp.float32),
    )(embT, denseT, linT, w1e, w1d, b1r, w2, b2r, w3t, dwc, c0)


def kernel(sparse, dense, embed_tables, linear_tables, dense_w, bias,
           W1, b1, W2, b2, W3, b3):
    tab2d = jnp.transpose(embed_tables, (0, 2, 1)).reshape(FD, V)
    idxT = jnp.transpose(sparse).astype(jnp.int32)
    denseT = jnp.transpose(dense)
    embT, linT = _sc_gather(tab2d, linear_tables, idxT)
    c0 = (bias + b3[0]).reshape(1, 1)
    return _tc_forward(
        embT, denseT, linT,
        W1[:FD], W1[FD:], b1.reshape(1, H), W2, b2.reshape(1, H),
        W3.reshape(1, H), dense_w.reshape(ND, 1), c0)
